# baseline (speedup 1.0000x reference)
import functools

import jax
import jax.numpy as jnp
from jax import lax
from jax.experimental import pallas as pl
from jax.experimental.pallas import tpu as pltpu
from jax.experimental.pallas import tpu_sc as plsc

NC = 2
NS = 16
NW = NC * NS
CHUNK = 128
K2 = 4
NBUF = 3
L = 2
SUP = K2 * CHUNK


def _sc_gather(tot, d, dtype):
    per_w = tot // NW
    n_chunks = per_w // CHUNK
    n_super = n_chunks // K2
    assert per_w * NW == tot and n_chunks * CHUNK == per_w
    assert n_super * K2 == n_chunks

    mesh = plsc.VectorSubcoreMesh(
        core_axis_name="c", subcore_axis_name="s",
        num_cores=NC, num_subcores=NS)

    @functools.partial(
        pl.kernel,
        out_type=jax.ShapeDtypeStruct((NW, per_w, d), dtype),
        mesh=mesh,
        scratch_types=[
            pltpu.VMEM((per_w,), jnp.int32),
            pltpu.VMEM((NBUF, SUP, d), dtype),
            [pltpu.SemaphoreType.DMA] * NBUF,
            [pltpu.SemaphoreType.DMA] * NBUF,
        ],
        compiler_params=pltpu.CompilerParams(use_tc_tiling_on_sc=False),
    )
    def run(tab_hbm, idx_hbm, out_hbm, idx_v, rows_v, gsem, ssem):
        wid = lax.axis_index("s") * NC + lax.axis_index("c")
        pltpu.sync_copy(idx_hbm.at[wid], idx_v)

        def fire_gather(g, b):
            pltpu.async_copy(
                tab_hbm.at[idx_v.at[pl.ds(g * SUP, SUP)]], rows_v.at[b],
                gsem[b])

        def wait_gather(g, b):
            pltpu.make_async_copy(
                tab_hbm.at[idx_v.at[pl.ds(g * SUP, SUP)]], rows_v.at[b],
                gsem[b]).wait()

        def fire_scatter(g, b):
            pltpu.async_copy(
                rows_v.at[b], out_hbm.at[wid, pl.ds(g * SUP, SUP)],
                ssem[b])

        def wait_scatter(b):
            pltpu.make_async_copy(
                rows_v.at[b], out_hbm.at[wid, pl.ds(0, SUP)],
                ssem[b]).wait()

        for g in range(L):
            fire_gather(g, g % NBUF)

        for g in range(NBUF - L):
            wait_gather(g, g)
            fire_scatter(g, g)
            fire_gather(g + L, (g + L) % NBUF)

        g_lo = NBUF - L
        g_hi = n_super - L
        n_steady = ((g_hi - g_lo) // NBUF) * NBUF
        @pl.loop(0, n_steady // NBUF)
        def _step(t):
            for u in range(NBUF):
                g = g_lo + t * NBUF + u
                b = (g_lo + u) % NBUF
                bn = (b + L) % NBUF
                wait_gather(g, b)
                fire_scatter(g, b)
                wait_scatter(bn)
                fire_gather(g + L, bn)

        for g in range(g_lo + n_steady, n_super):
            b = g % NBUF
            wait_gather(g, b)
            fire_scatter(g, b)
            if g + L < n_super:
                bn = (g + L) % NBUF
                wait_scatter(bn)
                fire_gather(g + L, bn)

        for b in range(min(NBUF, n_super)):
            wait_scatter(b)

    return run


def kernel(x, table):
    b, h = x.shape
    n, d = table.shape
    tot = b * h
    idx = x.reshape(NW, tot // NW).astype(jnp.int32)
    out = _sc_gather(tot, d, table.dtype)(table, idx)
    return out.reshape(b, h, d)


# speedup vs baseline: 1.3443x; 1.3443x over previous
import functools

import jax
import jax.numpy as jnp
from jax import lax
from jax.experimental import pallas as pl
from jax.experimental.pallas import tpu as pltpu
from jax.experimental.pallas import tpu_sc as plsc

NC = 2
NS = 16
NW = NC * NS
NBUF = 8
L = 6


def _sc_gather(nb, h, d, dtype):
    b_w = nb // NW
    hp = (h + 7) // 8 * 8
    dp = 128
    assert b_w * NW == nb and b_w % NBUF == 0 and b_w > NBUF

    mesh = plsc.VectorSubcoreMesh(
        core_axis_name="c", subcore_axis_name="s",
        num_cores=NC, num_subcores=NS)

    @functools.partial(
        pl.kernel,
        out_type=jax.ShapeDtypeStruct((nb, hp, dp), dtype),
        mesh=mesh,
        scratch_types=[
            pltpu.VMEM((b_w, h), jnp.int32),
            pltpu.VMEM((NBUF, h, d), dtype),
            [pltpu.SemaphoreType.DMA] * NBUF,
            [pltpu.SemaphoreType.DMA] * NBUF,
        ],
        compiler_params=pltpu.CompilerParams(use_tc_tiling_on_sc=False),
    )
    def run(tab_hbm, idx_hbm, out_hbm, idx_v, bufs, gsem, ssem):
        wid = lax.axis_index("s") * NC + lax.axis_index("c")
        b0 = wid * b_w
        pltpu.sync_copy(idx_hbm.at[pl.ds(b0, b_w)], idx_v)

        def fire_gather(j, bf):
            pltpu.async_copy(
                tab_hbm.at[idx_v.at[j]], bufs.at[bf], gsem[bf])

        def wait_gather(j, bf):
            pltpu.make_async_copy(
                tab_hbm.at[idx_v.at[j]], bufs.at[bf], gsem[bf]).wait()

        def fire_out(j, bf):
            pltpu.async_copy(
                bufs.at[bf],
                out_hbm.at[b0 + j, pl.ds(0, h), pl.ds(0, d)],
                ssem[bf])

        def wait_out(bf):
            pltpu.make_async_copy(
                bufs.at[bf],
                out_hbm.at[0, pl.ds(0, h), pl.ds(0, d)],
                ssem[bf]).wait()

        for j in range(L):
            fire_gather(j, j % NBUF)

        for j in range(NBUF - L):
            wait_gather(j, j)
            fire_out(j, j)
            fire_gather(j + L, (j + L) % NBUF)

        g_lo = NBUF - L
        g_hi = b_w - L
        n_steady = ((g_hi - g_lo) // NBUF) * NBUF
        @pl.loop(0, n_steady // NBUF)
        def _step(t):
            for u in range(NBUF):
                j = g_lo + t * NBUF + u
                bf = (g_lo + u) % NBUF
                bn = (bf + L) % NBUF
                wait_gather(j, bf)
                fire_out(j, bf)
                wait_out(bn)
                fire_gather(j + L, bn)

        for j in range(g_lo + n_steady, b_w):
            bf = j % NBUF
            wait_gather(j, bf)
            fire_out(j, bf)
            if j + L < b_w:
                bn = (j + L) % NBUF
                wait_out(bn)
                fire_gather(j + L, bn)

        for bf in range(min(NBUF, b_w)):
            wait_out(bf)

    return run


def kernel(x, table):
    nb, h = x.shape
    n, d = table.shape
    xi = x.astype(jnp.int32)
    q = _sc_gather(nb, h, d, table.dtype)(table, xi)
    return q[:, :h, :d]
